# Initial kernel scaffold; baseline (speedup 1.0000x reference)
import jax
import jax.numpy as jnp
from jax.experimental import pallas as pl


def kernel(features, adj, Ws, As, W_out, a_out):
    raise NotImplementedError("write your pallas kernel here")


# speedup vs baseline: 2925.7948x; 2925.7948x over previous
import functools

import jax
import jax.numpy as jnp
from jax.experimental import pallas as pl
from jax.experimental.pallas import tpu as pltpu

_ALPHA = 0.2


def _prologue1_body(nheads, nhid, x_ref, w_ref, asrc_ref, adst_ref,
                    haug_ref, fsrc_ref, fdst_ref):
    h = jnp.dot(x_ref[...], w_ref[...], preferred_element_type=jnp.float32)
    fsrc_ref[...] = jnp.dot(h, asrc_ref[...], preferred_element_type=jnp.float32)
    fdst_ref[...] = jnp.dot(h, adst_ref[...], preferred_element_type=jnp.float32)
    w = nhid + 1
    ones = jnp.ones((h.shape[0], 1), jnp.float32)
    for hd in range(nheads):
        haug_ref[:, hd * w:hd * w + nhid] = h[:, hd * nhid:(hd + 1) * nhid]
        haug_ref[:, hd * w + nhid:hd * w + nhid + 1] = ones


def _attn1_body(nheads, nhid, nj, fsrc_ref, fdstT_ref, haug_ref, adj_ref,
                x1_ref, acc_ref):
    j = pl.program_id(1)
    w = nhid + 1

    @pl.when(j == 0)
    def _init():
        acc_ref[...] = jnp.zeros_like(acc_ref)

    adjf = (adj_ref[...] != 0).astype(jnp.float32)
    haug = haug_ref[...]
    for hd in range(nheads):
        s = fsrc_ref[:, hd:hd + 1] + fdstT_ref[hd:hd + 1, :]
        leak = jnp.where(s >= 0.0, s, _ALPHA * s)
        p = jnp.exp(-leak) * adjf
        acc_ref[:, hd * w:(hd + 1) * w] += jnp.dot(
            p, haug[:, hd * w:(hd + 1) * w], preferred_element_type=jnp.float32)

    @pl.when(j == nj - 1)
    def _fin():
        acc = acc_ref[...]
        for hd in range(nheads):
            hp = acc[:, hd * w:hd * w + nhid]
            rs = acc[:, hd * w + nhid:hd * w + nhid + 1]
            x = hp / rs
            x1_ref[:, hd * nhid:(hd + 1) * nhid] = jnp.where(
                x > 0.0, x, jnp.exp(x) - 1.0)


def _prologue2_body(nclass, x_ref, w_ref, a2_ref, haug_ref, f2_ref):
    h2 = jnp.dot(x_ref[...], w_ref[...], preferred_element_type=jnp.float32)
    f2_ref[...] = jnp.dot(h2, a2_ref[...], preferred_element_type=jnp.float32)
    haug_ref[:, 0:nclass] = h2
    haug_ref[:, nclass:nclass + 1] = jnp.ones((h2.shape[0], 1), jnp.float32)


def _attn2_body(nclass, nj, f2_ref, f2T_ref, haug_ref, adj_ref, out_ref,
                acc_ref):
    j = pl.program_id(1)

    @pl.when(j == 0)
    def _init():
        acc_ref[...] = jnp.zeros_like(acc_ref)

    adjf = (adj_ref[...] != 0).astype(jnp.float32)
    s = f2_ref[:, 0:1] + f2T_ref[1:2, :]
    leak = jnp.where(s >= 0.0, s, _ALPHA * s)
    p = jnp.exp(-leak) * adjf
    acc_ref[...] += jnp.dot(p, haug_ref[...], preferred_element_type=jnp.float32)

    @pl.when(j == nj - 1)
    def _fin():
        acc = acc_ref[...]
        x = acc[:, 0:nclass] / acc[:, nclass:nclass + 1]
        x = jnp.where(x > 0.0, x, jnp.exp(x) - 1.0)
        m = jnp.max(x, axis=1, keepdims=True)
        lse = m + jnp.log(jnp.sum(jnp.exp(x - m), axis=1, keepdims=True))
        out_ref[...] = x - lse


def kernel(features, adj, Ws, As, W_out, a_out):
    n, nfeat = features.shape
    nheads, _, nhid = Ws.shape
    nclass = W_out.shape[1]
    ndim = nheads * nhid
    w1 = nhid + 1

    BI, BJ = 256, 512
    ni, nj = n // BI, n // BJ

    W_all = jnp.transpose(Ws, (1, 0, 2)).reshape(nfeat, ndim)
    eye = jnp.eye(nheads, dtype=jnp.float32)
    a_src_mat = (eye[:, None, :] * As[:, 0, :nhid][:, :, None]).reshape(ndim, nheads)
    a_dst_mat = (eye[:, None, :] * As[:, 0, nhid:][:, :, None]).reshape(ndim, nheads)
    a2_mat = jnp.pad(
        jnp.stack([a_out[0, :nclass], a_out[0, nclass:]], axis=1),
        ((0, 0), (0, 6)))

    haug, fsrc, fdst = pl.pallas_call(
        functools.partial(_prologue1_body, nheads, nhid),
        grid=(ni,),
        in_specs=[
            pl.BlockSpec((BI, nfeat), lambda i: (i, 0)),
            pl.BlockSpec((nfeat, ndim), lambda i: (0, 0)),
            pl.BlockSpec((ndim, nheads), lambda i: (0, 0)),
            pl.BlockSpec((ndim, nheads), lambda i: (0, 0)),
        ],
        out_specs=[
            pl.BlockSpec((BI, nheads * w1), lambda i: (i, 0)),
            pl.BlockSpec((BI, nheads), lambda i: (i, 0)),
            pl.BlockSpec((BI, nheads), lambda i: (i, 0)),
        ],
        out_shape=[
            jax.ShapeDtypeStruct((n, nheads * w1), jnp.float32),
            jax.ShapeDtypeStruct((n, nheads), jnp.float32),
            jax.ShapeDtypeStruct((n, nheads), jnp.float32),
        ],
    )(features, W_all, a_src_mat, a_dst_mat)

    fdstT = fdst.T

    x1 = pl.pallas_call(
        functools.partial(_attn1_body, nheads, nhid, nj),
        grid=(ni, nj),
        in_specs=[
            pl.BlockSpec((BI, nheads), lambda i, j: (i, 0)),
            pl.BlockSpec((nheads, BJ), lambda i, j: (0, j)),
            pl.BlockSpec((BJ, nheads * w1), lambda i, j: (j, 0)),
            pl.BlockSpec((BI, BJ), lambda i, j: (i, j)),
        ],
        out_specs=pl.BlockSpec((BI, ndim), lambda i, j: (i, 0)),
        out_shape=jax.ShapeDtypeStruct((n, ndim), jnp.float32),
        scratch_shapes=[pltpu.VMEM((BI, nheads * w1), jnp.float32)],
        compiler_params=pltpu.CompilerParams(
            dimension_semantics=("parallel", "arbitrary")),
    )(fsrc, fdstT, haug, adj)

    h2aug, f2 = pl.pallas_call(
        functools.partial(_prologue2_body, nclass),
        grid=(ni,),
        in_specs=[
            pl.BlockSpec((BI, ndim), lambda i: (i, 0)),
            pl.BlockSpec((ndim, nclass), lambda i: (0, 0)),
            pl.BlockSpec((nclass, 8), lambda i: (0, 0)),
        ],
        out_specs=[
            pl.BlockSpec((BI, nclass + 1), lambda i: (i, 0)),
            pl.BlockSpec((BI, 8), lambda i: (i, 0)),
        ],
        out_shape=[
            jax.ShapeDtypeStruct((n, nclass + 1), jnp.float32),
            jax.ShapeDtypeStruct((n, 8), jnp.float32),
        ],
    )(x1, W_out, a2_mat)

    f2T = f2.T

    out = pl.pallas_call(
        functools.partial(_attn2_body, nclass, nj),
        grid=(ni, nj),
        in_specs=[
            pl.BlockSpec((BI, 8), lambda i, j: (i, 0)),
            pl.BlockSpec((8, BJ), lambda i, j: (0, j)),
            pl.BlockSpec((BJ, nclass + 1), lambda i, j: (j, 0)),
            pl.BlockSpec((BI, BJ), lambda i, j: (i, j)),
        ],
        out_specs=pl.BlockSpec((BI, nclass), lambda i, j: (i, 0)),
        out_shape=jax.ShapeDtypeStruct((n, nclass), jnp.float32),
        scratch_shapes=[pltpu.VMEM((BI, nclass + 1), jnp.float32)],
        compiler_params=pltpu.CompilerParams(
            dimension_semantics=("parallel", "arbitrary")),
    )(f2, f2T, h2aug, adj)

    return out
